# Initial kernel scaffold; baseline (speedup 1.0000x reference)
import jax
import jax.numpy as jnp
from jax.experimental import pallas as pl


def kernel(src_wid, src_iob, src_pos, src_ner, W_word, W_iob, W_pos, W_ner, W_resize, b_resize):
    raise NotImplementedError("write your pallas kernel here")


# speedup vs baseline: 5.9492x; 5.9492x over previous
import functools

import jax
import jax.numpy as jnp
from jax import lax
from jax.experimental import pallas as pl
from jax.experimental.pallas import tpu as pltpu
from jax.experimental.pallas import tpu_sc as plsc

B, L = 1024, 200
V_WORD, D_WORD = 100000, 300
V_IOB, D_IOB = 5, 16
V_POS, D_POS = 50, 16
V_NER, D_NER = 20, 16
D_MODEL = 512
N_TOK = B * L

_ROWS_BLK = 2000


def _proj_body(w_ref, wr_ref, b_ref, out_ref):
    out_ref[...] = (
        jnp.dot(w_ref[...], wr_ref[...], preferred_element_type=jnp.float32)
        + b_ref[...]
    )


def _comb_body(iob_ref, pos_ref, ner_ref, wri_ref, wrp_ref, wrn_ref, out_ref):
    iob_p = jnp.dot(iob_ref[...], wri_ref[...], preferred_element_type=jnp.float32)
    pos_p = jnp.dot(pos_ref[...], wrp_ref[...], preferred_element_type=jnp.float32)
    ner_p = jnp.dot(ner_ref[...], wrn_ref[...], preferred_element_type=jnp.float32)
    full = (
        iob_p[:, None, None, :]
        + pos_p[None, :, None, :]
        + ner_p[None, None, :, :]
    )
    out_ref[...] = full.reshape(V_IOB * V_POS * V_NER, D_MODEL)


def _build_tables(W_word, W_iob, W_pos, W_ner, W_resize, b_resize):
    wr_word = W_resize[:D_WORD]
    wr_iob = W_resize[D_WORD:D_WORD + D_IOB]
    wr_pos = W_resize[D_WORD + D_IOB:D_WORD + D_IOB + D_POS]
    wr_ner = W_resize[D_WORD + D_IOB + D_POS:]

    wp = pl.pallas_call(
        _proj_body,
        grid=(V_WORD // _ROWS_BLK,),
        in_specs=[
            pl.BlockSpec((_ROWS_BLK, D_WORD), lambda i: (i, 0)),
            pl.BlockSpec((D_WORD, D_MODEL), lambda i: (0, 0)),
            pl.BlockSpec((1, D_MODEL), lambda i: (0, 0)),
        ],
        out_specs=pl.BlockSpec((_ROWS_BLK, D_MODEL), lambda i: (i, 0)),
        out_shape=jax.ShapeDtypeStruct((V_WORD, D_MODEL), jnp.float32),
    )(W_word, wr_word, b_resize.reshape(1, D_MODEL))

    comb = pl.pallas_call(
        _comb_body,
        out_shape=jax.ShapeDtypeStruct((V_IOB * V_POS * V_NER, D_MODEL), jnp.float32),
    )(W_iob, W_pos, W_ner, wr_iob, wr_pos, wr_ner)
    return wp, comb


_NC, _NS, _LN = 2, 16, 16
_NW = _NC * _NS
_RW = N_TOK // _NW
_CHUNK = 64


@functools.cache
def _make_sc_lookup():
    mesh = plsc.VectorSubcoreMesh(core_axis_name="c", subcore_axis_name="s",
                                  num_cores=_NC, num_subcores=_NS)

    @functools.partial(
        pl.kernel,
        out_type=jax.ShapeDtypeStruct((N_TOK, D_MODEL), jnp.float32),
        mesh=mesh,
        scratch_types=[
            pltpu.VMEM((_CHUNK,), jnp.int32),
            pltpu.VMEM((_CHUNK,), jnp.int32),
            pltpu.VMEM((_CHUNK,), jnp.int32),
            pltpu.VMEM((_CHUNK,), jnp.int32),
            pltpu.VMEM((_CHUNK,), jnp.int32),
            pltpu.VMEM((_CHUNK, D_MODEL), jnp.float32),
            pltpu.VMEM((_CHUNK, D_MODEL), jnp.float32),
            pltpu.SemaphoreType.DMA,
        ],
    )
    def _sc_lookup(wid_hbm, iob_hbm, pos_hbm, ner_hbm, wp_hbm, comb_hbm,
                   out_hbm, widv, iobv, posv, nerv, cidxv, rows, rows2, sem):
        w = lax.axis_index("s") * _NC + lax.axis_index("c")
        base = w * _RW

        def step(i, carry):
            off = base + i * _CHUNK
            pltpu.sync_copy(wid_hbm.at[pl.ds(off, _CHUNK)], widv)
            pltpu.sync_copy(iob_hbm.at[pl.ds(off, _CHUNK)], iobv)
            pltpu.sync_copy(pos_hbm.at[pl.ds(off, _CHUNK)], posv)
            pltpu.sync_copy(ner_hbm.at[pl.ds(off, _CHUNK)], nerv)
            for k in range(_CHUNK // _LN):
                s = pl.ds(k * _LN, _LN)
                cidxv[s] = iobv[s] * (V_POS * V_NER) + posv[s] * V_NER + nerv[s]
            cp1 = pltpu.async_copy(wp_hbm.at[widv], rows, sem)
            cp2 = pltpu.async_copy(comb_hbm.at[cidxv], rows2, sem)
            cp1.wait()
            cp2.wait()

            def add_row(r, c):
                for g in range(D_MODEL // _LN):
                    s = pl.ds(g * _LN, _LN)
                    rows[r, s] = rows[r, s] + rows2[r, s]
                return c

            lax.fori_loop(0, _CHUNK, add_row, 0)
            pltpu.sync_copy(rows, out_hbm.at[pl.ds(off, _CHUNK)])
            return carry

        lax.fori_loop(0, _RW // _CHUNK, step, 0)

    return _sc_lookup


def kernel(src_wid, src_iob, src_pos, src_ner, W_word, W_iob, W_pos, W_ner,
           W_resize, b_resize):
    wp, comb = _build_tables(W_word, W_iob, W_pos, W_ner, W_resize, b_resize)
    wid = src_wid.reshape(N_TOK).astype(jnp.int32)
    iob = src_iob.reshape(N_TOK).astype(jnp.int32)
    pos = src_pos.reshape(N_TOK).astype(jnp.int32)
    ner = src_ner.reshape(N_TOK).astype(jnp.int32)
    out = _make_sc_lookup()(wid, iob, pos, ner, wp, comb)
    return out.reshape(B, L, D_MODEL)
